# baseline (speedup 1.0000x reference)
import functools

import jax
import jax.numpy as jnp
from jax import lax
from jax.experimental import pallas as pl
from jax.experimental.pallas import tpu as pltpu
from jax.experimental.pallas import tpu_sc as plsc

N_NODES = 10000
N_EDGES = 320000
HIDDEN = 128
VOCAB = 780
N_TREES = 256
N_ITERS = 3

_BLK = 1000
_GRID = N_NODES // _BLK

_SUBC = 16
_EPT = N_EDGES // _SUBC
_K = 40
_CHUNKS = _EPT // _K
_RPT = 624


def _matmul_body(x_ref, w_ref, o_ref):
    o_ref[...] = jnp.dot(x_ref[...], w_ref[...], preferred_element_type=jnp.float32)


def _matmul(x, w):
    n, k = x.shape
    m = w.shape[1]
    return pl.pallas_call(
        _matmul_body,
        grid=(_GRID,),
        in_specs=[
            pl.BlockSpec((_BLK, k), lambda i: (i, 0)),
            pl.BlockSpec((k, m), lambda i: (0, 0)),
        ],
        out_specs=pl.BlockSpec((_BLK, m), lambda i: (i, 0)),
        out_shape=jax.ShapeDtypeStruct((n, m), jnp.float32),
    )(x, w)


def _node_step_body(x_ref, nm_ref, nrm_ref, wz1, wz2, wh1, wh2, ur, bz, bh, br,
                    m_out, b_out):
    x = x_ref[...]
    nm = nm_ref[...]
    nrm = nrm_ref[...]
    f32 = jnp.float32
    z = jax.nn.sigmoid(jnp.dot(x, wz1[...], preferred_element_type=f32)
                       + jnp.dot(nm, wz2[...], preferred_element_type=f32)
                       + bz[...])
    t = jnp.tanh(jnp.dot(x, wh1[...], preferred_element_type=f32)
                 + jnp.dot(nrm, wh2[...], preferred_element_type=f32)
                 + bh[...])
    m_n = (1.0 - z) * nm + z * t
    m_out[...] = m_n
    b_out[...] = jnp.dot(m_n, ur[...], preferred_element_type=f32) + br[...]


def _node_step(x, node_m, node_rm, wz1, wz2, wh1, wh2, ur, bz, bh, br):
    blk = lambda i: (i, 0)
    full = lambda i: (0, 0)
    return pl.pallas_call(
        _node_step_body,
        grid=(_GRID,),
        in_specs=[
            pl.BlockSpec((_BLK, HIDDEN), blk),
            pl.BlockSpec((_BLK, HIDDEN), blk),
            pl.BlockSpec((_BLK, HIDDEN), blk),
            pl.BlockSpec((HIDDEN, HIDDEN), full),
            pl.BlockSpec((HIDDEN, HIDDEN), full),
            pl.BlockSpec((HIDDEN, HIDDEN), full),
            pl.BlockSpec((HIDDEN, HIDDEN), full),
            pl.BlockSpec((HIDDEN, HIDDEN), full),
            pl.BlockSpec((1, HIDDEN), full),
            pl.BlockSpec((1, HIDDEN), full),
            pl.BlockSpec((1, HIDDEN), full),
        ],
        out_specs=[
            pl.BlockSpec((_BLK, HIDDEN), blk),
            pl.BlockSpec((_BLK, HIDDEN), blk),
        ],
        out_shape=[
            jax.ShapeDtypeStruct((N_NODES, HIDDEN), jnp.float32),
            jax.ShapeDtypeStruct((N_NODES, HIDDEN), jnp.float32),
        ],
    )(x, node_m, node_rm, wz1, wz2, wh1, wh2, ur, bz, bh, br)


_QPAD = 896


def _readout_body(x_ref, h_ref, tv_ref, wid_ref, pt_ref,
                  w1, w2, bw, wo, bo, u1, u2, u3, bu, us, bs, acc_ref):
    i = pl.program_id(0)
    f32 = jnp.float32
    x = x_ref[...]
    h = h_ref[...]
    tv = tv_ref[...]
    qh = jax.nn.relu(jnp.dot(h, w1[...], preferred_element_type=f32)
                     + jnp.dot(tv, w2[...], preferred_element_type=f32)
                     + bw[...])
    q = jnp.dot(qh, wo[...], preferred_element_type=f32) + bo[...]
    rowmax = jnp.max(q, axis=1, keepdims=True)
    lse = jnp.log(jnp.sum(jnp.exp(q - rowmax), axis=1, keepdims=True)) + rowmax
    wid = wid_ref[...]
    cols = lax.broadcasted_iota(jnp.int32, q.shape, 1)
    sel = jnp.sum(jnp.where(cols == wid, q, 0.0), axis=1, keepdims=True)
    q_loss = jnp.sum(lse - sel)
    q_hit = jnp.sum((sel == rowmax).astype(f32))
    ph = jax.nn.relu(jnp.dot(x, u1[...], preferred_element_type=f32)
                     + jnp.dot(h, u2[...], preferred_element_type=f32)
                     + jnp.dot(tv, u3[...], preferred_element_type=f32)
                     + bu[...])
    p = jnp.dot(ph, us[...], preferred_element_type=f32) + bs[...]
    pt = pt_ref[...].astype(f32)
    p_loss = jnp.sum(jnp.maximum(p, 0.0) - p * pt
                     + jnp.log1p(jnp.exp(-jnp.abs(p))))
    p_hit = jnp.sum(((p > 0.0).astype(f32) == pt).astype(f32))
    rows8 = lax.broadcasted_iota(jnp.int32, (8, 128), 0)
    cols8 = lax.broadcasted_iota(jnp.int32, (8, 128), 1)
    part = jnp.where((rows8 == 0) & (cols8 == 0), q_loss, 0.0)
    part = part + jnp.where((rows8 == 0) & (cols8 == 1), p_loss, 0.0)
    part = part + jnp.where((rows8 == 0) & (cols8 == 2), q_hit, 0.0)
    part = part + jnp.where((rows8 == 0) & (cols8 == 3), p_hit, 0.0)

    @pl.when(i == 0)
    def _():
        acc_ref[...] = jnp.zeros((8, 128), f32)

    acc_ref[...] += part


def _readout(x, h, tvp, wid2, pt2, w1, w2, bw, wo, bo, u1, u2, u3, bu, us, bs):
    blk = lambda i: (i, 0)
    full = lambda i: (0, 0)
    return pl.pallas_call(
        _readout_body,
        grid=(_GRID,),
        in_specs=[
            pl.BlockSpec((_BLK, HIDDEN), blk),
            pl.BlockSpec((_BLK, HIDDEN), blk),
            pl.BlockSpec((_BLK, HIDDEN), blk),
            pl.BlockSpec((_BLK, 1), blk),
            pl.BlockSpec((_BLK, 1), blk),
            pl.BlockSpec((HIDDEN, HIDDEN), full),
            pl.BlockSpec((HIDDEN, HIDDEN), full),
            pl.BlockSpec((1, HIDDEN), full),
            pl.BlockSpec((HIDDEN, _QPAD), full),
            pl.BlockSpec((1, _QPAD), full),
            pl.BlockSpec((HIDDEN, HIDDEN), full),
            pl.BlockSpec((HIDDEN, HIDDEN), full),
            pl.BlockSpec((HIDDEN, HIDDEN), full),
            pl.BlockSpec((1, HIDDEN), full),
            pl.BlockSpec((HIDDEN, 1), full),
            pl.BlockSpec((1, 1), full),
        ],
        out_specs=pl.BlockSpec((8, 128), full),
        out_shape=jax.ShapeDtypeStruct((8, 128), jnp.float32),
    )(x, h, tvp, wid2, pt2, w1, w2, bw, wo, bo, u1, u2, u3, bu, us, bs)


def _edge_body(m_hbm, b_hbm, a_hbm, pairs_hbm,
               nm_out, nrm_out,
               ibuf, m_v, b_v, a_v, acc,
               semi0, semi1, semi2, semm0, semm1, semb0, semb1, sema0, sema1):
    c = lax.axis_index("c")
    s = lax.axis_index("s")
    semi = (semi0, semi1, semi2)
    semm = (semm0, semm1)
    semb = (semb0, semb1)
    sema = (sema0, sema1)

    zero = jnp.zeros((16,), jnp.float32)

    def zrow(k, _):
        for g in range(8):
            m_v[0, k, pl.ds(g * 16, 16)] = zero
        return 0

    lax.fori_loop(0, _K, zrow, 0)

    base = s * _RPT

    def _row_chunks(fn):
        for j in range(15):
            fn(base + j * _K, _K)

        @pl.when(s < _SUBC - 1)
        def _():
            fn(base + 15 * _K, 24)

        @pl.when(s == _SUBC - 1)
        def _():
            fn(base + 15 * _K, _K)

    _row_chunks(lambda off, r: pltpu.sync_copy(m_v.at[0].at[pl.ds(0, r)],
                                               acc.at[pl.ds(off, r)]))

    def issue_idx(j, i3):
        pltpu.async_copy(pairs_hbm.at[s].at[j], ibuf.at[i3], semi[i3])

    def issue_gathers(b, i3):
        pltpu.async_copy(m_hbm.at[ibuf.at[i3, 0]], m_v.at[b], semm[b])

        @pl.when(c == 1)
        def _():
            pltpu.async_copy(b_hbm.at[ibuf.at[i3, 0]], b_v.at[b], semb[b])
            pltpu.async_copy(a_hbm.at[ibuf.at[i3, 1]], a_v.at[b], sema[b])

    def wait_idx(i3):
        pltpu.make_async_copy(pairs_hbm.at[s].at[0], ibuf.at[i3],
                              semi[i3]).wait()

    def wait_gathers(b, i3):
        pltpu.make_async_copy(m_hbm.at[ibuf.at[i3, 0]], m_v.at[b],
                              semm[b]).wait()

        @pl.when(c == 1)
        def _():
            pltpu.make_async_copy(b_hbm.at[ibuf.at[i3, 0]], b_v.at[b],
                                  semb[b]).wait()
            pltpu.make_async_copy(a_hbm.at[ibuf.at[i3, 1]], a_v.at[b],
                                  sema[b]).wait()

    def body(j, u):
        jb = u % 2
        ib = u % 3
        last = isinstance(j, int) and j + 1 >= _CHUNKS
        if not (isinstance(j, int) and j + 2 >= _CHUNKS):
            issue_idx(j + 2, (ib + 2) % 3)
        if not last:
            wait_idx((ib + 1) % 3)
            issue_gathers(1 - jb, (ib + 1) % 3)
        wait_gathers(jb, ib)

        @pl.when(c == 1)
        def _():
            def ebody(k, _):
                for g in range(8):
                    sl = pl.ds(g * 16, 16)
                    av = a_v[jb, k, sl]
                    bv = b_v[jb, k, sl]
                    mv = m_v[jb, k, sl]
                    m_v[jb, k, sl] = mv / (1.0 + jnp.exp(-(av + bv)))
                return 0

            lax.fori_loop(0, _K, ebody, 0, unroll=2)

        pltpu.sync_copy(m_v.at[jb], acc.at[ibuf.at[ib, 1]], add=True)

    issue_idx(0, 0)
    issue_idx(1, 1)
    wait_idx(0)
    issue_gathers(0, 0)
    plsc.subcore_barrier()

    def outer(t, _):
        j0 = t * 6
        for u in range(6):
            body(j0 + u, u)
        return 0

    lax.fori_loop(0, (_CHUNKS - 2) // 6, outer, 0)
    body(_CHUNKS - 2, (_CHUNKS - 2) % 6)
    body(_CHUNKS - 1, (_CHUNKS - 1) % 6)
    plsc.subcore_barrier()

    def _writeback(off, r):
        @pl.when(c == 0)
        def _():
            pltpu.sync_copy(acc.at[pl.ds(off, r)], nm_out.at[pl.ds(off, r)])

        @pl.when(c == 1)
        def _():
            pltpu.sync_copy(acc.at[pl.ds(off, r)], nrm_out.at[pl.ds(off, r)])

    _row_chunks(_writeback)


@functools.cache
def _make_edge_pass():
    return functools.partial(
        pl.kernel,
        out_type=[
            jax.ShapeDtypeStruct((N_NODES, HIDDEN), jnp.float32),
            jax.ShapeDtypeStruct((N_NODES, HIDDEN), jnp.float32),
        ],
        mesh=plsc.VectorSubcoreMesh(core_axis_name="c", subcore_axis_name="s"),
        scratch_types=[
            pltpu.VMEM((3, 2, _K), jnp.int32),
            pltpu.VMEM((2, _K, HIDDEN), jnp.float32),
            pltpu.VMEM((2, _K, HIDDEN), jnp.float32),
            pltpu.VMEM((2, _K, HIDDEN), jnp.float32),
            pltpu.VMEM_SHARED((N_NODES, HIDDEN), jnp.float32),
        ] + [pltpu.SemaphoreType.DMA] * 9,
    )(_edge_body)


def _edge_pass(m_tab, b_tab, a_tab, pairs):
    return _make_edge_pass()(m_tab, b_tab, a_tab, pairs)


def kernel(wid, edge_index, node_tree, p_targets, tree_vec, emb, W_z, b_z,
           W_r, U_r, b_r, W_h, b_h, W, b_W, U, b_U, W_o, b_o, U_s, b_s):
    f32 = jnp.float32
    H = HIDDEN
    pairs = jnp.stack(
        [edge_index[0].reshape(_SUBC, _CHUNKS, _K),
         edge_index[1].reshape(_SUBC, _CHUNKS, _K)], axis=2
    ).astype(jnp.int32)
    x = jnp.take(emb, wid, axis=0)

    wz1, wz2 = W_z[:H], W_z[H:]
    wh1, wh2 = W_h[:H], W_h[H:]
    bz = b_z.reshape(1, H)
    bh = b_h.reshape(1, H)
    br = b_r.reshape(1, H)
    a_tab = _matmul(x, W_r)

    node_m = jnp.zeros((N_NODES, H), f32)
    node_rm = jnp.zeros((N_NODES, H), f32)
    for _ in range(N_ITERS):
        m_tab, b_tab = _node_step(x, node_m, node_rm, wz1, wz2, wh1, wh2,
                                  U_r, bz, bh, br)
        node_m, node_rm = _edge_pass(m_tab, b_tab, a_tab, pairs)
    h = node_m

    tv = jnp.take(tree_vec, node_tree, axis=0)
    tvp = jnp.pad(tv, ((0, 0), (0, H - tv.shape[1])))

    w1 = W[:H]
    w2 = jnp.pad(W[H:], ((0, H - (W.shape[0] - H)), (0, 0)))
    u1 = U[:H]
    u2 = U[H:2 * H]
    u3 = jnp.pad(U[2 * H:], ((0, H - (U.shape[0] - 2 * H)), (0, 0)))
    wo = jnp.pad(W_o, ((0, 0), (0, _QPAD - VOCAB)))
    bo = jnp.concatenate([b_o, jnp.full((_QPAD - VOCAB,), -1e30, f32)]).reshape(1, _QPAD)
    bw = b_W.reshape(1, H)
    bu = b_U.reshape(1, H)
    bs = b_s.reshape(1, 1)

    sums = _readout(x, h, tvp, wid.reshape(-1, 1).astype(jnp.int32),
                    p_targets.reshape(-1, 1).astype(jnp.int32),
                    w1, w2, bw, wo, bo, u1, u2, u3, bu, U_s, bs)
    q_loss = sums[0, 0] / N_TREES
    p_loss = sums[0, 1] / N_TREES
    q_acc = sums[0, 2] / N_NODES
    p_acc = sums[0, 3] / N_NODES
    return (q_loss, p_loss, q_acc, p_acc)


# speedup vs baseline: 5.8529x; 5.8529x over previous
import functools

import jax
import jax.numpy as jnp
from jax import lax
from jax.experimental import pallas as pl
from jax.experimental.pallas import tpu as pltpu
from jax.experimental.pallas import tpu_sc as plsc

N_NODES = 10000
N_EDGES = 320000
HIDDEN = 128
VOCAB = 780
N_TREES = 256
N_ITERS = 3

_BLK = 1000
_GRID = N_NODES // _BLK

_SUBC = 16
_EPT = N_EDGES // _SUBC
_K = 40
_CHUNKS = _EPT // _K
_RPT = 624


def _matmul_body(x_ref, w_ref, o_ref):
    o_ref[...] = jnp.dot(x_ref[...], w_ref[...], preferred_element_type=jnp.float32)


def _matmul(x, w):
    n, k = x.shape
    m = w.shape[1]
    return pl.pallas_call(
        _matmul_body,
        grid=(_GRID,),
        in_specs=[
            pl.BlockSpec((_BLK, k), lambda i: (i, 0)),
            pl.BlockSpec((k, m), lambda i: (0, 0)),
        ],
        out_specs=pl.BlockSpec((_BLK, m), lambda i: (i, 0)),
        out_shape=jax.ShapeDtypeStruct((n, m), jnp.float32),
    )(x, w)


def _node_step_body(x_ref, nm_ref, nrm_ref, wz1, wz2, wh1, wh2, ur, bz, bh, br,
                    m_out, b_out):
    x = x_ref[...]
    nm = nm_ref[...]
    nrm = nrm_ref[...]
    f32 = jnp.float32
    z = jax.nn.sigmoid(jnp.dot(x, wz1[...], preferred_element_type=f32)
                       + jnp.dot(nm, wz2[...], preferred_element_type=f32)
                       + bz[...])
    t = jnp.tanh(jnp.dot(x, wh1[...], preferred_element_type=f32)
                 + jnp.dot(nrm, wh2[...], preferred_element_type=f32)
                 + bh[...])
    m_n = (1.0 - z) * nm + z * t
    m_out[...] = m_n
    b_out[...] = jnp.dot(m_n, ur[...], preferred_element_type=f32) + br[...]


def _node_step(x, node_m, node_rm, wz1, wz2, wh1, wh2, ur, bz, bh, br):
    blk = lambda i: (i, 0)
    full = lambda i: (0, 0)
    return pl.pallas_call(
        _node_step_body,
        grid=(_GRID,),
        in_specs=[
            pl.BlockSpec((_BLK, HIDDEN), blk),
            pl.BlockSpec((_BLK, HIDDEN), blk),
            pl.BlockSpec((_BLK, HIDDEN), blk),
            pl.BlockSpec((HIDDEN, HIDDEN), full),
            pl.BlockSpec((HIDDEN, HIDDEN), full),
            pl.BlockSpec((HIDDEN, HIDDEN), full),
            pl.BlockSpec((HIDDEN, HIDDEN), full),
            pl.BlockSpec((HIDDEN, HIDDEN), full),
            pl.BlockSpec((1, HIDDEN), full),
            pl.BlockSpec((1, HIDDEN), full),
            pl.BlockSpec((1, HIDDEN), full),
        ],
        out_specs=[
            pl.BlockSpec((_BLK, HIDDEN), blk),
            pl.BlockSpec((_BLK, HIDDEN), blk),
        ],
        out_shape=[
            jax.ShapeDtypeStruct((N_NODES, HIDDEN), jnp.float32),
            jax.ShapeDtypeStruct((N_NODES, HIDDEN), jnp.float32),
        ],
    )(x, node_m, node_rm, wz1, wz2, wh1, wh2, ur, bz, bh, br)


_QPAD = 896


def _readout_body(x_ref, h_ref, tv_ref, wid_ref, pt_ref,
                  w1, w2, bw, wo, bo, u1, u2, u3, bu, us, bs, acc_ref):
    i = pl.program_id(0)
    f32 = jnp.float32
    x = x_ref[...]
    h = h_ref[...]
    tv = tv_ref[...]
    qh = jax.nn.relu(jnp.dot(h, w1[...], preferred_element_type=f32)
                     + jnp.dot(tv, w2[...], preferred_element_type=f32)
                     + bw[...])
    q = jnp.dot(qh, wo[...], preferred_element_type=f32) + bo[...]
    rowmax = jnp.max(q, axis=1, keepdims=True)
    lse = jnp.log(jnp.sum(jnp.exp(q - rowmax), axis=1, keepdims=True)) + rowmax
    wid = wid_ref[...]
    cols = lax.broadcasted_iota(jnp.int32, q.shape, 1)
    sel = jnp.sum(jnp.where(cols == wid, q, 0.0), axis=1, keepdims=True)
    q_loss = jnp.sum(lse - sel)
    q_hit = jnp.sum((sel == rowmax).astype(f32))
    ph = jax.nn.relu(jnp.dot(x, u1[...], preferred_element_type=f32)
                     + jnp.dot(h, u2[...], preferred_element_type=f32)
                     + jnp.dot(tv, u3[...], preferred_element_type=f32)
                     + bu[...])
    p = jnp.dot(ph, us[...], preferred_element_type=f32) + bs[...]
    pt = pt_ref[...].astype(f32)
    p_loss = jnp.sum(jnp.maximum(p, 0.0) - p * pt
                     + jnp.log1p(jnp.exp(-jnp.abs(p))))
    p_hit = jnp.sum(((p > 0.0).astype(f32) == pt).astype(f32))
    rows8 = lax.broadcasted_iota(jnp.int32, (8, 128), 0)
    cols8 = lax.broadcasted_iota(jnp.int32, (8, 128), 1)
    part = jnp.where((rows8 == 0) & (cols8 == 0), q_loss, 0.0)
    part = part + jnp.where((rows8 == 0) & (cols8 == 1), p_loss, 0.0)
    part = part + jnp.where((rows8 == 0) & (cols8 == 2), q_hit, 0.0)
    part = part + jnp.where((rows8 == 0) & (cols8 == 3), p_hit, 0.0)

    @pl.when(i == 0)
    def _():
        acc_ref[...] = jnp.zeros((8, 128), f32)

    acc_ref[...] += part


def _readout(x, h, tvp, wid2, pt2, w1, w2, bw, wo, bo, u1, u2, u3, bu, us, bs):
    blk = lambda i: (i, 0)
    full = lambda i: (0, 0)
    return pl.pallas_call(
        _readout_body,
        grid=(_GRID,),
        in_specs=[
            pl.BlockSpec((_BLK, HIDDEN), blk),
            pl.BlockSpec((_BLK, HIDDEN), blk),
            pl.BlockSpec((_BLK, HIDDEN), blk),
            pl.BlockSpec((_BLK, 1), blk),
            pl.BlockSpec((_BLK, 1), blk),
            pl.BlockSpec((HIDDEN, HIDDEN), full),
            pl.BlockSpec((HIDDEN, HIDDEN), full),
            pl.BlockSpec((1, HIDDEN), full),
            pl.BlockSpec((HIDDEN, _QPAD), full),
            pl.BlockSpec((1, _QPAD), full),
            pl.BlockSpec((HIDDEN, HIDDEN), full),
            pl.BlockSpec((HIDDEN, HIDDEN), full),
            pl.BlockSpec((HIDDEN, HIDDEN), full),
            pl.BlockSpec((1, HIDDEN), full),
            pl.BlockSpec((HIDDEN, 1), full),
            pl.BlockSpec((1, 1), full),
        ],
        out_specs=pl.BlockSpec((8, 128), full),
        out_shape=jax.ShapeDtypeStruct((8, 128), jnp.float32),
    )(x, h, tvp, wid2, pt2, w1, w2, bw, wo, bo, u1, u2, u3, bu, us, bs)


def _edge_body(m_hbm, b_hbm, a_hbm, pairs_hbm,
               nm_out, nrm_out,
               ibuf, m_v, b_v, a_v, acc,
               semi0, semi1, semi2, semm0, semm1, semm2,
               semb0, semb1, semb2, sema0, sema1, sema2):
    c = lax.axis_index("c")
    s = lax.axis_index("s")
    semi = (semi0, semi1, semi2)
    semm = (semm0, semm1, semm2)
    semb = (semb0, semb1, semb2)
    sema = (sema0, sema1, sema2)

    zero = jnp.zeros((16,), jnp.float32)

    def zrow(k, _):
        for g in range(8):
            m_v[0, k, pl.ds(g * 16, 16)] = zero
        return 0

    lax.fori_loop(0, _K, zrow, 0)

    base = s * _RPT

    def _row_chunks(fn):
        for j in range(15):
            fn(base + j * _K, _K)

        @pl.when(s < _SUBC - 1)
        def _():
            fn(base + 15 * _K, 24)

        @pl.when(s == _SUBC - 1)
        def _():
            fn(base + 15 * _K, _K)

    _row_chunks(lambda off, r: pltpu.sync_copy(m_v.at[0].at[pl.ds(0, r)],
                                               acc.at[pl.ds(off, r)]))

    def issue_idx(j, i3):
        pltpu.async_copy(pairs_hbm.at[s].at[j], ibuf.at[i3], semi[i3])

    def issue_gathers(b, i3):
        pltpu.async_copy(m_hbm.at[ibuf.at[i3, 0]], m_v.at[b], semm[b])

        @pl.when(c == 1)
        def _():
            pltpu.async_copy(b_hbm.at[ibuf.at[i3, 0]], b_v.at[b], semb[b])
            pltpu.async_copy(a_hbm.at[ibuf.at[i3, 1]], a_v.at[b], sema[b])

    def wait_idx(i3):
        pltpu.make_async_copy(pairs_hbm.at[s].at[0], ibuf.at[i3],
                              semi[i3]).wait()

    def wait_gathers(b, i3):
        pltpu.make_async_copy(m_hbm.at[ibuf.at[i3, 0]], m_v.at[b],
                              semm[b]).wait()

        @pl.when(c == 1)
        def _():
            pltpu.make_async_copy(b_hbm.at[ibuf.at[i3, 0]], b_v.at[b],
                                  semb[b]).wait()
            pltpu.make_async_copy(a_hbm.at[ibuf.at[i3, 1]], a_v.at[b],
                                  sema[b]).wait()

    def body(j, u):
        jb = u
        ib = u
        last = isinstance(j, int) and j + 1 >= _CHUNKS
        if not (isinstance(j, int) and j + 2 >= _CHUNKS):
            issue_idx(j + 2, (ib + 2) % 3)
        if not last:
            wait_idx((ib + 1) % 3)
            issue_gathers((jb + 1) % 3, (ib + 1) % 3)
        wait_gathers(jb, ib)

        @pl.when(c == 1)
        def _():
            def ebody(k, _):
                for g in range(8):
                    sl = pl.ds(g * 16, 16)
                    av = a_v[jb, k, sl]
                    bv = b_v[jb, k, sl]
                    mv = m_v[jb, k, sl]
                    m_v[jb, k, sl] = mv / (1.0 + jnp.exp(-(av + bv)))
                return 0

            lax.fori_loop(0, _K, ebody, 0)

        pltpu.sync_copy(m_v.at[jb], acc.at[ibuf.at[ib, 1]], add=True)

    issue_idx(0, 0)
    issue_idx(1, 1)
    wait_idx(0)
    issue_gathers(0, 0)
    plsc.subcore_barrier()

    def outer(t, _):
        j0 = t * 3
        for u in range(3):
            body(j0 + u, u)
        return 0

    lax.fori_loop(0, (_CHUNKS - 2) // 3, outer, 0)
    body(_CHUNKS - 2, (_CHUNKS - 2) % 3)
    body(_CHUNKS - 1, (_CHUNKS - 1) % 3)
    plsc.subcore_barrier()

    def _writeback(off, r):
        @pl.when(c == 0)
        def _():
            pltpu.sync_copy(acc.at[pl.ds(off, r)], nm_out.at[pl.ds(off, r)])

        @pl.when(c == 1)
        def _():
            pltpu.sync_copy(acc.at[pl.ds(off, r)], nrm_out.at[pl.ds(off, r)])

    _row_chunks(_writeback)


@functools.cache
def _make_edge_pass():
    return functools.partial(
        pl.kernel,
        out_type=[
            jax.ShapeDtypeStruct((N_NODES, HIDDEN), jnp.float32),
            jax.ShapeDtypeStruct((N_NODES, HIDDEN), jnp.float32),
        ],
        mesh=plsc.VectorSubcoreMesh(core_axis_name="c", subcore_axis_name="s"),
        scratch_types=[
            pltpu.VMEM((3, 2, _K), jnp.int32),
            pltpu.VMEM((3, _K, HIDDEN), jnp.float32),
            pltpu.VMEM((3, _K, HIDDEN), jnp.float32),
            pltpu.VMEM((3, _K, HIDDEN), jnp.float32),
            pltpu.VMEM_SHARED((N_NODES, HIDDEN), jnp.float32),
        ] + [pltpu.SemaphoreType.DMA] * 12,
    )(_edge_body)


def _edge_pass(m_tab, b_tab, a_tab, pairs):
    return _make_edge_pass()(m_tab, b_tab, a_tab, pairs)


def kernel(wid, edge_index, node_tree, p_targets, tree_vec, emb, W_z, b_z,
           W_r, U_r, b_r, W_h, b_h, W, b_W, U, b_U, W_o, b_o, U_s, b_s):
    f32 = jnp.float32
    H = HIDDEN
    pairs = jnp.stack(
        [edge_index[0].reshape(_SUBC, _CHUNKS, _K),
         edge_index[1].reshape(_SUBC, _CHUNKS, _K)], axis=2
    ).astype(jnp.int32)
    x = jnp.take(emb, wid, axis=0)

    wz1, wz2 = W_z[:H], W_z[H:]
    wh1, wh2 = W_h[:H], W_h[H:]
    bz = b_z.reshape(1, H)
    bh = b_h.reshape(1, H)
    br = b_r.reshape(1, H)
    a_tab = _matmul(x, W_r)

    node_m = jnp.zeros((N_NODES, H), f32)
    node_rm = jnp.zeros((N_NODES, H), f32)
    for _ in range(N_ITERS):
        m_tab, b_tab = _node_step(x, node_m, node_rm, wz1, wz2, wh1, wh2,
                                  U_r, bz, bh, br)
        node_m, node_rm = _edge_pass(m_tab, b_tab, a_tab, pairs)
    h = node_m

    tv = jnp.take(tree_vec, node_tree, axis=0)
    tvp = jnp.pad(tv, ((0, 0), (0, H - tv.shape[1])))

    w1 = W[:H]
    w2 = jnp.pad(W[H:], ((0, H - (W.shape[0] - H)), (0, 0)))
    u1 = U[:H]
    u2 = U[H:2 * H]
    u3 = jnp.pad(U[2 * H:], ((0, H - (U.shape[0] - 2 * H)), (0, 0)))
    wo = jnp.pad(W_o, ((0, 0), (0, _QPAD - VOCAB)))
    bo = jnp.concatenate([b_o, jnp.full((_QPAD - VOCAB,), -1e30, f32)]).reshape(1, _QPAD)
    bw = b_W.reshape(1, H)
    bu = b_U.reshape(1, H)
    bs = b_s.reshape(1, 1)

    sums = _readout(x, h, tvp, wid.reshape(-1, 1).astype(jnp.int32),
                    p_targets.reshape(-1, 1).astype(jnp.int32),
                    w1, w2, bw, wo, bo, u1, u2, u3, bu, U_s, bs)
    q_loss = sums[0, 0] / N_TREES
    p_loss = sums[0, 1] / N_TREES
    q_acc = sums[0, 2] / N_NODES
    p_acc = sums[0, 3] / N_NODES
    return (q_loss, p_loss, q_acc, p_acc)
